# baseline (speedup 1.0000x reference)
import jax
import jax.numpy as jnp
from jax.experimental import pallas as pl
from jax.experimental.pallas import tpu as pltpu

B, T, C = 64, 577, 768
ABLATION_VALUE = 10000000.0
INF = float("inf")
NR = 8
NW = 6


def _fused_kernel(idx_ref, x_hbm, out_hbm, v_ref, rbuf, wbuf,
                  fb_s, mb_s, sfb_s, v_s, rsem, wsem):
    def read_start(b):
        slot = jax.lax.rem(b, NR)
        pltpu.make_async_copy(x_hbm.at[b], rbuf.at[slot], rsem.at[slot]).start()

    def read_wait(b):
        slot = jax.lax.rem(b, NR)
        pltpu.make_async_copy(x_hbm.at[b], rbuf.at[slot], rsem.at[slot]).wait()

    def write_start(b):
        slot = jax.lax.rem(b, NW)
        pltpu.make_async_copy(wbuf.at[slot], out_hbm.at[b], wsem.at[slot]).start()

    def write_wait(b):
        slot = jax.lax.rem(b, NW)
        pltpu.make_async_copy(wbuf.at[slot], out_hbm.at[b], wsem.at[slot]).wait()

    def compute(b):
        slot = jax.lax.rem(b, NR)
        wslot = jax.lax.rem(b, NW)
        xb = rbuf[slot]
        rowmins = jnp.min(xb, axis=1, keepdims=True)
        fb_s[b] = jnp.min(rowmins)
        idx = idx_ref[b]
        tids = jax.lax.broadcasted_iota(jnp.int32, (T, 1), 0)
        mb_s[b] = jnp.min(jnp.where(tids == idx, INF, rowmins))
        wbuf[wslot] = xb.T

    for b in range(NR):
        read_start(b)

    def body1(b, _):
        read_wait(b)
        compute(b)
        write_start(b)
        read_start(b + NR)
        return 0

    jax.lax.fori_loop(0, NW, body1, 0)

    def body2(b, _):
        write_wait(b - NW)
        read_wait(b)
        compute(b)
        write_start(b)
        read_start(b + NR)
        return 0

    jax.lax.fori_loop(NW, B - NR, body2, 0)

    def body3(b, _):
        write_wait(b - NW)
        read_wait(b)
        compute(b)
        write_start(b)
        return 0

    jax.lax.fori_loop(B - NR, B, body3, 0)

    def drain(b, _):
        write_wait(b)
        return 0

    jax.lax.fori_loop(B - NW, B, drain, 0)

    def bwd(t, carry):
        i = B - 1 - t
        carry = jnp.minimum(carry, fb_s[i])
        sfb_s[i] = carry
        return carry

    jax.lax.fori_loop(0, B, bwd, jnp.float32(INF))

    def fwd(i, carry):
        pmb, vmin = carry
        m = jnp.minimum(jnp.minimum(pmb, sfb_s[i]), vmin)
        v = jnp.where(m == 0.0, jnp.float32(0.0), m - ABLATION_VALUE)
        v_s[i] = v
        return jnp.minimum(pmb, mb_s[i]), jnp.minimum(vmin, v)

    jax.lax.fori_loop(0, B, fwd, (jnp.float32(INF), jnp.float32(INF)))

    def wr(i, _):
        v_ref[pl.ds(i, 1), :] = jnp.full((1, 128), v_s[i], jnp.float32)
        return 0

    jax.lax.fori_loop(0, B, wr, 0)


def _rmw_kernel(idx_ref, v_ref, in_ref, out_ref):
    j = pl.program_id(0)
    idx = idx_ref[j]
    qa = (idx // 128) * 128
    lid = jax.lax.broadcasted_iota(jnp.int32, (1, C, 128), 2)
    out_ref[...] = jnp.where(lid == idx - qa, v_ref[j], in_ref[...])


def kernel(x, indices):
    out1, v_pad = pl.pallas_call(
        _fused_kernel,
        grid_spec=pltpu.PrefetchScalarGridSpec(
            num_scalar_prefetch=1,
            grid=(1,),
            in_specs=[pl.BlockSpec(memory_space=pl.ANY)],
            out_specs=[
                pl.BlockSpec(memory_space=pl.ANY),
                pl.BlockSpec((B, 128), lambda i, idx_ref: (0, 0)),
            ],
            scratch_shapes=[
                pltpu.VMEM((NR, T, C), jnp.float32),
                pltpu.VMEM((NW, C, T), jnp.float32),
                pltpu.SMEM((B,), jnp.float32),
                pltpu.SMEM((B,), jnp.float32),
                pltpu.SMEM((B,), jnp.float32),
                pltpu.SMEM((B,), jnp.float32),
                pltpu.SemaphoreType.DMA((NR,)),
                pltpu.SemaphoreType.DMA((NW,)),
            ],
        ),
        out_shape=[
            jax.ShapeDtypeStruct((B, C, T), jnp.float32),
            jax.ShapeDtypeStruct((B, 128), jnp.float32),
        ],
    )(indices, x)
    v = v_pad[:, 0]

    out = pl.pallas_call(
        _rmw_kernel,
        grid_spec=pltpu.PrefetchScalarGridSpec(
            num_scalar_prefetch=2,
            grid=(B,),
            in_specs=[
                pl.BlockSpec(
                    (1, C, 128),
                    lambda j, idx_ref, v_ref: (j, 0, idx_ref[j] // 128),
                )
            ],
            out_specs=pl.BlockSpec(
                (1, C, 128),
                lambda j, idx_ref, v_ref: (j, 0, idx_ref[j] // 128),
            ),
        ),
        out_shape=jax.ShapeDtypeStruct((B, C, T), jnp.float32),
        input_output_aliases={2: 0},
    )(indices, v, out1)
    return out


# speedup vs baseline: 1.0185x; 1.0185x over previous
import jax
import jax.numpy as jnp
from jax.experimental import pallas as pl
from jax.experimental.pallas import tpu as pltpu

B, T, C = 64, 577, 768
ABLATION_VALUE = 10000000.0
INF = float("inf")
BB = 8
BT = 4


def _mins_kernel(idx_ref, x_ref, v_ref, fb_s, mb_s, sfb_s, v_s):
    j = pl.program_id(0)
    xb = x_ref[...]
    rowmins = jnp.min(xb, axis=2)
    tids = jax.lax.broadcasted_iota(jnp.int32, (BB, T), 1)
    for k in range(BB):
        b = j * BB + k
        idx = idx_ref[b]
        fb_s[b] = jnp.min(rowmins[k])
        mb_s[b] = jnp.min(jnp.where(tids[k] == idx, INF, rowmins[k]))

    @pl.when(j == (B // BB) - 1)
    def _():
        def bwd(t, carry):
            i = B - 1 - t
            carry = jnp.minimum(carry, fb_s[i])
            sfb_s[i] = carry
            return carry

        jax.lax.fori_loop(0, B, bwd, jnp.float32(INF))

        def fwd(i, carry):
            pmb, vmin = carry
            m = jnp.minimum(jnp.minimum(pmb, sfb_s[i]), vmin)
            v = jnp.where(m == 0.0, jnp.float32(0.0), m - ABLATION_VALUE)
            v_s[i] = v
            return jnp.minimum(pmb, mb_s[i]), jnp.minimum(vmin, v)

        jax.lax.fori_loop(0, B, fwd, (jnp.float32(INF), jnp.float32(INF)))

        def wr(i, _):
            v_ref[pl.ds(i, 1), :] = jnp.full((1, 128), v_s[i], jnp.float32)
            return 0

        jax.lax.fori_loop(0, B, wr, 0)


def _transpose_kernel(idx_ref, v_ref, x_ref, out_ref):
    j = pl.program_id(0)
    xt = jnp.transpose(x_ref[...], (0, 2, 1))
    tcol = jax.lax.broadcasted_iota(jnp.int32, (BT, C, T), 2)
    idxs = jnp.concatenate(
        [jnp.full((1, 1, 1), idx_ref[j * BT + k], jnp.int32) for k in range(BT)], 0
    )
    vals = jnp.concatenate(
        [jnp.full((1, 1, 1), v_ref[j * BT + k], jnp.float32) for k in range(BT)], 0
    )
    out_ref[...] = jnp.where(tcol == idxs, vals, xt)


def kernel(x, indices):
    v_pad = pl.pallas_call(
        _mins_kernel,
        grid_spec=pltpu.PrefetchScalarGridSpec(
            num_scalar_prefetch=1,
            grid=(B // BB,),
            in_specs=[pl.BlockSpec((BB, T, C), lambda j, idx_ref: (j, 0, 0))],
            out_specs=pl.BlockSpec((B, 128), lambda j, idx_ref: (0, 0)),
            scratch_shapes=[
                pltpu.SMEM((B,), jnp.float32),
                pltpu.SMEM((B,), jnp.float32),
                pltpu.SMEM((B,), jnp.float32),
                pltpu.SMEM((B,), jnp.float32),
            ],
        ),
        out_shape=jax.ShapeDtypeStruct((B, 128), jnp.float32),
    )(indices, x)
    v = v_pad[:, 0]

    out = pl.pallas_call(
        _transpose_kernel,
        grid_spec=pltpu.PrefetchScalarGridSpec(
            num_scalar_prefetch=2,
            grid=(B // BT,),
            in_specs=[pl.BlockSpec((BT, T, C), lambda j, *_: (j, 0, 0))],
            out_specs=pl.BlockSpec((BT, C, T), lambda j, *_: (j, 0, 0)),
        ),
        out_shape=jax.ShapeDtypeStruct((B, C, T), jnp.float32),
    )(indices, v, x)
    return out
